# baseline (speedup 1.0000x reference)
import functools

import jax
import jax.numpy as jnp
from jax import lax
from jax.experimental import pallas as pl
from jax.experimental.pallas import tpu as pltpu
from jax.experimental.pallas import tpu_sc as plsc

_BATCH = 16384
_EMB = 64
_W = 2 * _EMB

_NC = 2
_NS = 16
_NW = _NC * _NS
_BPW = _BATCH // _NW
_CH = 128
_NCHUNK = _BPW // _CH


def _gather_body(u_hbm, v_hbm, user_hbm, item_hbm, urows_out, vrows_out,
                 uidx, vidx, rows, sem):
    wid = lax.axis_index("s") * _NC + lax.axis_index("c")
    base = wid * _BPW
    pltpu.sync_copy(u_hbm.at[wid], uidx)
    pltpu.sync_copy(v_hbm.at[wid], vidx)
    ucopies = [pltpu.async_copy(
        user_hbm.at[uidx.at[j]], rows.at[pl.ds(j * _CH, _CH)], sem)
        for j in range(_NCHUNK)]
    for c in ucopies:
        c.wait()
    pltpu.sync_copy(rows, urows_out.at[pl.ds(base, _BPW)])
    vcopies = [pltpu.async_copy(
        item_hbm.at[vidx.at[j]], rows.at[pl.ds(j * _CH, _CH)], sem)
        for j in range(_NCHUNK)]
    for c in vcopies:
        c.wait()
    pltpu.sync_copy(rows, vrows_out.at[pl.ds(base, _BPW)])


_sc_gather = functools.partial(
    pl.kernel,
    out_type=[jax.ShapeDtypeStruct((_BATCH, _W), jnp.float32),
              jax.ShapeDtypeStruct((_BATCH, _W), jnp.float32)],
    mesh=plsc.VectorSubcoreMesh(core_axis_name="c", subcore_axis_name="s"),
    scratch_types=[
        pltpu.VMEM((_NCHUNK, _CH), jnp.int32),
        pltpu.VMEM((_NCHUNK, _CH), jnp.int32),
        pltpu.VMEM((_BPW, _W), jnp.float32),
        pltpu.SemaphoreType.DMA,
    ],
)(_gather_body)


_BM = 2048


def _mlp_body(ug_ref, vg_ref, mu_ref, mv_ref, w1u_ref, w1v_ref, b1_ref,
              w2_ref, b2_ref, o_ref):
    ug = ug_ref[...]
    vg = vg_ref[...]
    zu = jnp.maximum(jnp.where(mu_ref[...] > 0.5, ug[:, _EMB:], ug[:, :_EMB]),
                     0.0)
    zv = jnp.maximum(jnp.where(mv_ref[...] > 0.5, vg[:, _EMB:], vg[:, :_EMB]),
                     0.0)
    h = (jnp.dot(zu, w1u_ref[...], preferred_element_type=jnp.float32)
         + jnp.dot(zv, w1v_ref[...], preferred_element_type=jnp.float32)
         + b1_ref[...])
    h = jnp.maximum(h, 0.0)
    o_ref[...] = (jnp.dot(h, w2_ref[...], preferred_element_type=jnp.float32)
                  + b2_ref[...])


def _mlp(ug, vg, mu, mv, w1u, w1v, b1, w2, b2):
    return pl.pallas_call(
        _mlp_body,
        grid=(_BATCH // _BM,),
        in_specs=[
            pl.BlockSpec((_BM, _W), lambda i: (i, 0)),
            pl.BlockSpec((_BM, _W), lambda i: (i, 0)),
            pl.BlockSpec((_BM, 1), lambda i: (i, 0)),
            pl.BlockSpec((_BM, 1), lambda i: (i, 0)),
            pl.BlockSpec((_EMB, _EMB), lambda i: (0, 0)),
            pl.BlockSpec((_EMB, _EMB), lambda i: (0, 0)),
            pl.BlockSpec((1, _EMB), lambda i: (0, 0)),
            pl.BlockSpec((_EMB, 1), lambda i: (0, 0)),
            pl.BlockSpec((1, 1), lambda i: (0, 0)),
        ],
        out_specs=pl.BlockSpec((_BM, 1), lambda i: (i, 0)),
        out_shape=jax.ShapeDtypeStruct((_BATCH, 1), jnp.float32),
    )(ug, vg, mu, mv, w1u, w1v, b1, w2, b2)


def kernel(u, v, user_emb, item_emb, W1, b1, W2, b2):
    u = u.astype(jnp.int32)
    v = v.astype(jnp.int32)
    u3 = (u // 2).reshape(_NW, _NCHUNK, _CH)
    v3 = (v // 2).reshape(_NW, _NCHUNK, _CH)
    mu = (u % 2).astype(jnp.float32).reshape(-1, 1)
    mv = (v % 2).astype(jnp.float32).reshape(-1, 1)
    ue2 = user_emb.reshape(-1, _W)
    ve2 = item_emb.reshape(-1, _W)
    ug, vg = _sc_gather(u3, v3, ue2, ve2)
    return _mlp(ug, vg, mu, mv, W1[:_EMB], W1[_EMB:], b1.reshape(1, _EMB),
                W2, b2.reshape(1, 1))


# speedup vs baseline: 1.1790x; 1.1790x over previous
import functools

import jax
import jax.numpy as jnp
from jax import lax
from jax.experimental import pallas as pl
from jax.experimental.pallas import tpu as pltpu
from jax.experimental.pallas import tpu_sc as plsc

_BATCH = 16384
_EMB = 64
_W = 2 * _EMB
_ROWS = 1000000
_HALF = _ROWS // 2

_NC = 2
_NS = 16
_NW = _NC * _NS
_BPW = _BATCH // _NW
_CH = 128
_NCHUNK = _BPW // _CH

_PC = 2048
_PG = -(-_ROWS // _PC)


def _pack_body(x_ref, eye_ref, o_ref):
    t = jnp.dot(x_ref[...].T, eye_ref[...],
                preferred_element_type=jnp.float32)
    o_ref[...] = jnp.concatenate([t[:_PC // 2], t[_PC // 2:]], axis=1)


def _pack(tab_t, eye):
    return pl.pallas_call(
        _pack_body,
        grid=(_PG,),
        in_specs=[
            pl.BlockSpec((_EMB, _PC), lambda i: (0, i)),
            pl.BlockSpec((_EMB, _EMB), lambda i: (0, 0)),
        ],
        out_specs=pl.BlockSpec((_PC // 2, _W), lambda i: (i, 0)),
        out_shape=jax.ShapeDtypeStruct((_PG * (_PC // 2), _W), jnp.float32),
    )(tab_t, eye)


def _gather_body(i_hbm, tab_hbm, out_hbm, idx, rows, sem):
    wid = lax.axis_index("s") * _NC + lax.axis_index("c")
    base = wid * _BPW
    pltpu.sync_copy(i_hbm.at[wid], idx)
    copies = [pltpu.async_copy(
        tab_hbm.at[idx.at[j]], rows.at[pl.ds(j * _CH, _CH)], sem)
        for j in range(_NCHUNK)]
    for c in copies:
        c.wait()
    pltpu.sync_copy(rows, out_hbm.at[pl.ds(base, _BPW)])


_sc_gather = functools.partial(
    pl.kernel,
    out_type=jax.ShapeDtypeStruct((_BATCH, _W), jnp.float32),
    mesh=plsc.VectorSubcoreMesh(core_axis_name="c", subcore_axis_name="s"),
    scratch_types=[
        pltpu.VMEM((_NCHUNK, _CH), jnp.int32),
        pltpu.VMEM((_BPW, _W), jnp.float32),
        pltpu.SemaphoreType.DMA,
    ],
)(_gather_body)


_BM = 2048


def _mlp_body(ug_ref, vg_ref, mu_ref, mv_ref, w1u_ref, w1v_ref, b1_ref,
              w2_ref, b2_ref, o_ref):
    ug = ug_ref[...]
    vg = vg_ref[...]
    zu = jnp.maximum(jnp.where(mu_ref[...] > 0.5, ug[:, _EMB:], ug[:, :_EMB]),
                     0.0)
    zv = jnp.maximum(jnp.where(mv_ref[...] > 0.5, vg[:, _EMB:], vg[:, :_EMB]),
                     0.0)
    h = (jnp.dot(zu, w1u_ref[...], preferred_element_type=jnp.float32)
         + jnp.dot(zv, w1v_ref[...], preferred_element_type=jnp.float32)
         + b1_ref[...])
    h = jnp.maximum(h, 0.0)
    o_ref[...] = (jnp.dot(h, w2_ref[...], preferred_element_type=jnp.float32)
                  + b2_ref[...])


def _mlp(ug, vg, mu, mv, w1u, w1v, b1, w2, b2):
    return pl.pallas_call(
        _mlp_body,
        grid=(_BATCH // _BM,),
        in_specs=[
            pl.BlockSpec((_BM, _W), lambda i: (i, 0)),
            pl.BlockSpec((_BM, _W), lambda i: (i, 0)),
            pl.BlockSpec((_BM, 1), lambda i: (i, 0)),
            pl.BlockSpec((_BM, 1), lambda i: (i, 0)),
            pl.BlockSpec((_EMB, _EMB), lambda i: (0, 0)),
            pl.BlockSpec((_EMB, _EMB), lambda i: (0, 0)),
            pl.BlockSpec((1, _EMB), lambda i: (0, 0)),
            pl.BlockSpec((_EMB, 1), lambda i: (0, 0)),
            pl.BlockSpec((1, 1), lambda i: (0, 0)),
        ],
        out_specs=pl.BlockSpec((_BM, 1), lambda i: (i, 0)),
        out_shape=jax.ShapeDtypeStruct((_BATCH, 1), jnp.float32),
    )(ug, vg, mu, mv, w1u, w1v, b1, w2, b2)


def kernel(u, v, user_emb, item_emb, W1, b1, W2, b2):
    u = u.astype(jnp.int32)
    v = v.astype(jnp.int32)
    u3 = ((u // _PC) * (_PC // 2) + (u % (_PC // 2))).reshape(
        _NW, _NCHUNK, _CH)
    v3 = ((v // _PC) * (_PC // 2) + (v % (_PC // 2))).reshape(
        _NW, _NCHUNK, _CH)
    mu = ((u // (_PC // 2)) % 2).astype(jnp.float32).reshape(-1, 1)
    mv = ((v // (_PC // 2)) % 2).astype(jnp.float32).reshape(-1, 1)
    eye = jnp.eye(_EMB, dtype=jnp.float32)
    up = _pack(user_emb.T, eye)
    vp = _pack(item_emb.T, eye)
    ug = _sc_gather(u3, up)
    vg = _sc_gather(v3, vp)
    return _mlp(ug, vg, mu, mv, W1[:_EMB], W1[_EMB:], b1.reshape(1, _EMB),
                W2, b2.reshape(1, 1))
